# baseline (speedup 1.0000x reference)
import functools

import jax
import jax.numpy as jnp
from jax import lax
from jax.experimental import pallas as pl
from jax.experimental.pallas import tpu as pltpu
from jax.experimental.pallas import tpu_sc as plsc

N = 10000
NP = 10240
F = 32
NC, NS = 2, 16
NW = NC * NS
CH = 125
NCHUNK = 80
EPT = CH * NCHUNK
NBUF = 8
LEAD = 4
EPS = 1e-5

_mesh = plsc.VectorSubcoreMesh(core_axis_name="c", subcore_axis_name="s")
_sc_params = pltpu.CompilerParams(use_tc_tiling_on_sc=False)


@functools.partial(
    pl.kernel,
    out_type=jax.ShapeDtypeStruct((NC, NP), jnp.float32),
    mesh=_mesh,
    scratch_types=[
        pltpu.VMEM((EPT // 16, 16), jnp.int32),
        pltpu.VMEM((NP,), jnp.float32),
        pltpu.VMEM((NP // NS,), jnp.float32),
        pltpu.VMEM((NP // NS,), jnp.float32),
        pltpu.VMEM_SHARED((NS, NP), jnp.float32),
    ],
    compiler_params=pltpu.CompilerParams(use_tc_tiling_on_sc=False,
                                         needs_layout_passes=False),
)
def _deg_kernel(dst_hbm, out_hbm, dst_v, hist_v, accv, tmpv, acc2_sh):
    c = lax.axis_index("c")
    s = lax.axis_index("s")
    wid = c * NS + s
    pltpu.sync_copy(dst_hbm.at[wid], dst_v)

    z = jnp.zeros((16,), jnp.float32)

    def zero_body(i, carry):
        hist_v[pl.ds(i * 16, 16)] = z
        return carry

    lax.fori_loop(0, NP // 16, zero_body, 0)

    ones = jnp.full((16,), 1.0, jnp.float32)

    def acc_body(k, carry):
        plsc.addupdate_scatter(hist_v, [dst_v[k]], ones)
        return carry

    lax.fori_loop(0, EPT // 16, acc_body, 0)

    pltpu.sync_copy(hist_v, acc2_sh.at[s])
    plsc.subcore_barrier()

    rpt = NP // NS

    def zero2(i, carry):
        accv[pl.ds(i * 16, 16)] = z
        return carry

    lax.fori_loop(0, rpt // 16, zero2, 0)

    for t in range(NS):
        pltpu.sync_copy(acc2_sh.at[t, pl.ds(s * rpt, rpt)], tmpv)

        def add_body(i, carry):
            accv[pl.ds(i * 16, 16)] += tmpv[pl.ds(i * 16, 16)]
            return carry

        lax.fori_loop(0, rpt // 16, add_body, 0)

    pltpu.sync_copy(accv, out_hbm.at[c, pl.ds(s * rpt, rpt)])


@functools.partial(
    pl.kernel,
    out_type=jax.ShapeDtypeStruct((NC, NP, F), jnp.float32),
    mesh=_mesh,
    scratch_types=[
        pltpu.VMEM((NCHUNK, CH), jnp.int32),
        pltpu.VMEM((NCHUNK, CH), jnp.int32),
        pltpu.VMEM((NBUF, CH, F), jnp.float32),
        pltpu.VMEM_SHARED((NP, F), jnp.float32),
        pltpu.VMEM_SHARED((NP, F), jnp.float32),
        pltpu.SemaphoreType.DMA((NBUF,)),
        pltpu.SemaphoreType.DMA((NBUF,)),
    ],
    compiler_params=_sc_params,
)
def _prop_kernel(m_hbm, src_hbm, dst_hbm, zc_hbm, out_hbm,
                 src_v, dst_v, rows_v, acc_sh, tab_sh, gsem, ssem):
    c = lax.axis_index("c")
    s = lax.axis_index("s")
    wid = c * NS + s
    pltpu.sync_copy(src_hbm.at[wid], src_v)
    pltpu.sync_copy(dst_hbm.at[wid], dst_v)
    rpt = NP // NS
    pltpu.sync_copy(zc_hbm.at[pl.ds(s * rpt, rpt)], acc_sh.at[pl.ds(s * rpt, rpt)])
    pltpu.sync_copy(m_hbm.at[pl.ds(s * rpt, rpt)], tab_sh.at[pl.ds(s * rpt, rpt)])
    plsc.subcore_barrier()

    def issue(j, b):
        pltpu.async_copy(tab_sh.at[src_v.at[j]], rows_v.at[b], gsem.at[b])

    for b in range(LEAD):
        issue(b, b)

    def body(jj, carry):
        base = jj * NBUF
        for k in range(NBUF):
            j = base + k
            pltpu.make_async_copy(tab_sh.at[src_v.at[0]], rows_v.at[k],
                                  gsem.at[k]).wait()
            pltpu.async_copy(rows_v.at[k], acc_sh.at[dst_v.at[j]], ssem.at[k],
                             add=True)
            nxt = j + LEAD
            bn = (k + LEAD) % NBUF

            @pl.when(nxt < NCHUNK)
            def _():
                @pl.when(j >= NBUF - LEAD)
                def _():
                    pltpu.make_async_copy(rows_v.at[bn],
                                          acc_sh.at[dst_v.at[0]],
                                          ssem.at[bn]).wait()
                issue(nxt, bn)
        return carry

    lax.fori_loop(0, NCHUNK // NBUF, body, 0)
    for b in range(NBUF):
        pltpu.make_async_copy(rows_v.at[b], acc_sh.at[dst_v.at[0]],
                              ssem.at[b]).wait()
    plsc.subcore_barrier()

    @pl.when(s == 0)
    def _():
        pltpu.sync_copy(acc_sh, out_hbm.at[c])


NR = N // 4
NPR = NP // 4


def _tc_mm1(x_ref, w1_ref, t1_ref):
    t1_ref[...] = jnp.dot(x_ref[...], w1_ref[...],
                          preferred_element_type=jnp.float32)


def _tc_pre(t1_ref, dp_ref, sel_ref, m1_ref, dinv_ref):
    d4 = dp_ref[0, 0:NR, :] + dp_ref[1, 0:NR, :]
    deg4 = jnp.dot(d4, sel_ref[...], preferred_element_type=jnp.float32) + 1.0
    dinv = lax.rsqrt(deg4)
    dinv_ref[...] = dinv
    m1_ref[0:NR, :] = dinv * t1_ref[...]


def _bn_relu(p, fold_ref, ga, be):
    mu = jnp.dot(jnp.mean(p, axis=0, keepdims=True), fold_ref[...],
                 preferred_element_type=jnp.float32)
    var = jnp.dot(jnp.mean((p - mu) * (p - mu), axis=0, keepdims=True),
                  fold_ref[...], preferred_element_type=jnp.float32)
    return jnp.maximum((p - mu) * lax.rsqrt(var + EPS) * ga + be, 0.0)


def _tc_mid1(sp_ref, m1_ref, dinv_ref, fold_ref, b_ref, ga_ref, be_ref,
             w2_ref, m2_ref):
    dinv = dinv_ref[...]
    p = dinv * (sp_ref[0, 0:NR, :] + sp_ref[1, 0:NR, :] + m1_ref[0:NR, :]) \
        + b_ref[...]
    h = _bn_relu(p, fold_ref, ga_ref[...], be_ref[...])
    t2 = jnp.dot(h, w2_ref[...], preferred_element_type=jnp.float32)
    m2_ref[0:NR, :] = dinv * t2


def _tc_mid2(sp_ref, m2_ref, dinv_ref, fold_ref, b_ref, ga_ref, be_ref,
             m3_ref):
    dinv = dinv_ref[...]
    p = dinv * (sp_ref[0, 0:NR, :] + sp_ref[1, 0:NR, :] + m2_ref[0:NR, :]) \
        + b_ref[...]
    m3_ref[0:NR, :] = dinv * _bn_relu(p, fold_ref, ga_ref[...], be_ref[...])


def _tc_fin(sp_ref, m3_ref, dinv_ref, fold_ref, x_ref, w3_ref, b_ref, ga_ref,
            be_ref, o_ref):
    dinv = dinv_ref[...]
    q = dinv * (sp_ref[0, 0:NR, :] + sp_ref[1, 0:NR, :] + m3_ref[0:NR, :])
    t3 = jnp.dot(q, w3_ref[...], preferred_element_type=jnp.float32) + b_ref[...]
    mu = jnp.dot(jnp.mean(t3, axis=0, keepdims=True), fold_ref[...],
                 preferred_element_type=jnp.float32)
    var = jnp.dot(jnp.mean((t3 - mu) * (t3 - mu), axis=0, keepdims=True),
                  fold_ref[...], preferred_element_type=jnp.float32)
    bn = (t3 - mu) * lax.rsqrt(var + EPS) * ga_ref[...] + be_ref[...]
    o_ref[...] = jnp.maximum(bn + x_ref[...], 0.0)


def _sds(shape):
    return jax.ShapeDtypeStruct(shape, jnp.float32)


def kernel(x, ei, batch, W1, b1, g1, be1, W2, b2, g2, be2, W3, b3, g3, be3):
    del batch
    ei32 = ei.astype(jnp.int32)
    eir = ei32.reshape(2, NW, NCHUNK, CH)
    srcp, dstp = eir[0], eir[1]
    dst16 = ei32[1].reshape(NW, EPT // 16, 16)
    zc = jnp.zeros((NP, F), jnp.float32)

    eye4 = jnp.eye(4, dtype=jnp.float32)
    w1i = jnp.kron(eye4, W1)
    w2i = jnp.kron(eye4, W2)
    w3i = jnp.kron(eye4, W3)
    sel = jnp.kron(eye4, jnp.ones((1, F), jnp.float32))
    quarter = jnp.full((4, 4), 0.25, jnp.float32)
    fold128 = jnp.kron(quarter, jnp.eye(F, dtype=jnp.float32))
    fold512 = jnp.kron(quarter, jnp.eye(4 * F, dtype=jnp.float32))
    b1r, g1r, be1r = (jnp.tile(v, 4).reshape(1, 128) for v in (b1, g1, be1))
    b2r, g2r, be2r = (jnp.tile(v, 4).reshape(1, 128) for v in (b2, g2, be2))
    b3r, g3r, be3r = (jnp.tile(v, 4).reshape(1, 512) for v in (b3, g3, be3))

    degp = _deg_kernel(dst16)
    t1 = pl.pallas_call(_tc_mm1, out_shape=_sds((NR, 128)))(
        x.reshape(NR, 512), w1i)
    m1, dinv = pl.pallas_call(
        _tc_pre,
        out_shape=[_sds((NPR, 128)), _sds((NR, 128))],
    )(t1, degp.reshape(2, NPR, 4), sel)

    s1 = _prop_kernel(m1.reshape(NP, F), srcp, dstp, zc)
    m2 = pl.pallas_call(
        _tc_mid1,
        out_shape=_sds((NPR, 128)),
    )(s1.reshape(2, NPR, 128), m1, dinv, fold128, b1r, g1r, be1r, w2i)

    s2 = _prop_kernel(m2.reshape(NP, F), srcp, dstp, zc)
    m3 = pl.pallas_call(
        _tc_mid2,
        out_shape=_sds((NPR, 128)),
    )(s2.reshape(2, NPR, 128), m2, dinv, fold128, b2r, g2r, be2r)

    s3 = _prop_kernel(m3.reshape(NP, F), srcp, dstp, zc)
    out = pl.pallas_call(
        _tc_fin,
        out_shape=_sds((NR, 512)),
    )(s3.reshape(2, NPR, 128), m3, dinv, fold512, x.reshape(NR, 512), w3i,
      b3r, g3r, be3r)
    return out.reshape(N, 4 * F)


# speedup vs baseline: 1.0158x; 1.0158x over previous
import functools

import jax
import jax.numpy as jnp
from jax import lax
from jax.experimental import pallas as pl
from jax.experimental.pallas import tpu as pltpu
from jax.experimental.pallas import tpu_sc as plsc

N = 10000
NP = 10240
F = 32
NC, NS = 2, 16
NW = NC * NS
CH = 125
NCHUNK = 80
EPT = CH * NCHUNK
NBUF = 8
LEAD = 4
EPS = 1e-5

_mesh = plsc.VectorSubcoreMesh(core_axis_name="c", subcore_axis_name="s")
_sc_params = pltpu.CompilerParams(use_tc_tiling_on_sc=False)


@functools.partial(
    pl.kernel,
    out_type=jax.ShapeDtypeStruct((NC, NP, 16), jnp.float32),
    mesh=_mesh,
    scratch_types=[
        pltpu.VMEM((NCHUNK, CH), jnp.int32),
        pltpu.VMEM((CH, 16), jnp.float32),
        pltpu.VMEM_SHARED((NP, 16), jnp.float32),
        pltpu.SemaphoreType.DMA((4,)),
    ],
    compiler_params=_sc_params,
)
def _deg_kernel(dst_hbm, z16_hbm, out_hbm, dst_v, ones_v, acc_sh, dsem):
    c = lax.axis_index("c")
    s = lax.axis_index("s")
    wid = c * NS + s
    pltpu.sync_copy(dst_hbm.at[wid], dst_v)

    one = jnp.full((16,), 1.0, jnp.float32)

    def ones_body(i, carry):
        ones_v[i] = one
        return carry

    lax.fori_loop(0, CH, ones_body, 0)
    rpt = NP // NS
    pltpu.sync_copy(z16_hbm.at[pl.ds(s * rpt, rpt)], acc_sh.at[pl.ds(s * rpt, rpt)])
    plsc.subcore_barrier()

    def body(jj, carry):
        base = jj * 4
        for k in range(4):
            pltpu.async_copy(ones_v, acc_sh.at[dst_v.at[base + k]],
                             dsem.at[k], add=True)
        return carry

    lax.fori_loop(0, NCHUNK // 4, body, 0)

    def drain(jj, carry):
        for k in range(4):
            pltpu.make_async_copy(ones_v, acc_sh.at[dst_v.at[0]],
                                  dsem.at[k]).wait()
        return carry

    lax.fori_loop(0, NCHUNK // 4, drain, 0)
    plsc.subcore_barrier()

    @pl.when(s == 0)
    def _():
        pltpu.sync_copy(acc_sh, out_hbm.at[c])


@functools.partial(
    pl.kernel,
    out_type=jax.ShapeDtypeStruct((NC, NP, F), jnp.float32),
    mesh=_mesh,
    scratch_types=[
        pltpu.VMEM((NCHUNK, CH), jnp.int32),
        pltpu.VMEM((NCHUNK, CH), jnp.int32),
        pltpu.VMEM((NBUF, CH, F), jnp.float32),
        pltpu.VMEM_SHARED((NP, F), jnp.float32),
        pltpu.VMEM_SHARED((NP, F), jnp.float32),
        pltpu.SemaphoreType.DMA((NBUF,)),
        pltpu.SemaphoreType.DMA((NBUF,)),
    ],
    compiler_params=_sc_params,
)
def _prop_kernel(m_hbm, src_hbm, dst_hbm, zc_hbm, out_hbm,
                 src_v, dst_v, rows_v, acc_sh, tab_sh, gsem, ssem):
    c = lax.axis_index("c")
    s = lax.axis_index("s")
    wid = c * NS + s
    pltpu.sync_copy(src_hbm.at[wid], src_v)
    pltpu.sync_copy(dst_hbm.at[wid], dst_v)
    rpt = NP // NS
    pltpu.sync_copy(zc_hbm.at[pl.ds(s * rpt, rpt)], acc_sh.at[pl.ds(s * rpt, rpt)])
    pltpu.sync_copy(m_hbm.at[pl.ds(s * rpt, rpt)], tab_sh.at[pl.ds(s * rpt, rpt)])
    plsc.subcore_barrier()

    def issue(j, b):
        pltpu.async_copy(tab_sh.at[src_v.at[j]], rows_v.at[b], gsem.at[b])

    for b in range(LEAD):
        issue(b, b)

    def body(jj, carry):
        base = jj * NBUF
        for k in range(NBUF):
            j = base + k
            pltpu.make_async_copy(tab_sh.at[src_v.at[0]], rows_v.at[k],
                                  gsem.at[k]).wait()
            pltpu.async_copy(rows_v.at[k], acc_sh.at[dst_v.at[j]], ssem.at[k],
                             add=True)
            nxt = j + LEAD
            bn = (k + LEAD) % NBUF

            @pl.when(nxt < NCHUNK)
            def _():
                @pl.when(j >= NBUF - LEAD)
                def _():
                    pltpu.make_async_copy(rows_v.at[bn],
                                          acc_sh.at[dst_v.at[0]],
                                          ssem.at[bn]).wait()
                issue(nxt, bn)
        return carry

    lax.fori_loop(0, NCHUNK // NBUF, body, 0)
    for b in range(NBUF):
        pltpu.make_async_copy(rows_v.at[b], acc_sh.at[dst_v.at[0]],
                              ssem.at[b]).wait()
    plsc.subcore_barrier()

    @pl.when(s == 0)
    def _():
        pltpu.sync_copy(acc_sh, out_hbm.at[c])


NR = N // 4
NPR = NP // 4


def _tc_mm1(x_ref, w1_ref, t1_ref):
    t1_ref[...] = jnp.dot(x_ref[...], w1_ref[...],
                          preferred_element_type=jnp.float32)


def _tc_pre(t1_ref, dp_ref, sel_ref, m1_ref, dinv_ref):
    d64 = dp_ref[0, 0:NR, :] + dp_ref[1, 0:NR, :]
    deg4 = jnp.dot(d64, sel_ref[...], preferred_element_type=jnp.float32) + 1.0
    dinv = lax.rsqrt(deg4)
    dinv_ref[...] = dinv
    m1_ref[0:NR, :] = dinv * t1_ref[...]


def _bn_relu(p, fold_ref, ga, be):
    mu = jnp.dot(jnp.mean(p, axis=0, keepdims=True), fold_ref[...],
                 preferred_element_type=jnp.float32)
    var = jnp.dot(jnp.mean((p - mu) * (p - mu), axis=0, keepdims=True),
                  fold_ref[...], preferred_element_type=jnp.float32)
    return jnp.maximum((p - mu) * lax.rsqrt(var + EPS) * ga + be, 0.0)


def _tc_mid1(sp_ref, m1_ref, dinv_ref, fold_ref, b_ref, ga_ref, be_ref,
             w2_ref, m2_ref):
    dinv = dinv_ref[...]
    p = dinv * (sp_ref[0, 0:NR, :] + sp_ref[1, 0:NR, :] + m1_ref[0:NR, :]) \
        + b_ref[...]
    h = _bn_relu(p, fold_ref, ga_ref[...], be_ref[...])
    t2 = jnp.dot(h, w2_ref[...], preferred_element_type=jnp.float32)
    m2_ref[0:NR, :] = dinv * t2


def _tc_mid2(sp_ref, m2_ref, dinv_ref, fold_ref, b_ref, ga_ref, be_ref,
             m3_ref):
    dinv = dinv_ref[...]
    p = dinv * (sp_ref[0, 0:NR, :] + sp_ref[1, 0:NR, :] + m2_ref[0:NR, :]) \
        + b_ref[...]
    m3_ref[0:NR, :] = dinv * _bn_relu(p, fold_ref, ga_ref[...], be_ref[...])


def _tc_fin(sp_ref, m3_ref, dinv_ref, fold_ref, x_ref, w3_ref, b_ref, ga_ref,
            be_ref, o_ref):
    dinv = dinv_ref[...]
    q = dinv * (sp_ref[0, 0:NR, :] + sp_ref[1, 0:NR, :] + m3_ref[0:NR, :])
    t3 = jnp.dot(q, w3_ref[...], preferred_element_type=jnp.float32) + b_ref[...]
    mu = jnp.dot(jnp.mean(t3, axis=0, keepdims=True), fold_ref[...],
                 preferred_element_type=jnp.float32)
    var = jnp.dot(jnp.mean((t3 - mu) * (t3 - mu), axis=0, keepdims=True),
                  fold_ref[...], preferred_element_type=jnp.float32)
    bn = (t3 - mu) * lax.rsqrt(var + EPS) * ga_ref[...] + be_ref[...]
    o_ref[...] = jnp.maximum(bn + x_ref[...], 0.0)


def _sds(shape):
    return jax.ShapeDtypeStruct(shape, jnp.float32)


def kernel(x, ei, batch, W1, b1, g1, be1, W2, b2, g2, be2, W3, b3, g3, be3):
    del batch
    eir = ei.astype(jnp.int32).reshape(2, NW, NCHUNK, CH)
    srcp, dstp = eir[0], eir[1]
    zc = jnp.zeros((NP, F), jnp.float32)
    z16 = jnp.zeros((NP, 16), jnp.float32)

    eye4 = jnp.eye(4, dtype=jnp.float32)
    w1i = jnp.kron(eye4, W1)
    w2i = jnp.kron(eye4, W2)
    w3i = jnp.kron(eye4, W3)
    sel = jnp.kron(eye4, jnp.zeros((16, F), jnp.float32).at[0, :].set(1.0))
    quarter = jnp.full((4, 4), 0.25, jnp.float32)
    fold128 = jnp.kron(quarter, jnp.eye(F, dtype=jnp.float32))
    fold512 = jnp.kron(quarter, jnp.eye(4 * F, dtype=jnp.float32))
    b1r, g1r, be1r = (jnp.tile(v, 4).reshape(1, 128) for v in (b1, g1, be1))
    b2r, g2r, be2r = (jnp.tile(v, 4).reshape(1, 128) for v in (b2, g2, be2))
    b3r, g3r, be3r = (jnp.tile(v, 4).reshape(1, 512) for v in (b3, g3, be3))

    degp = _deg_kernel(dstp, z16)
    t1 = pl.pallas_call(_tc_mm1, out_shape=_sds((NR, 128)))(
        x.reshape(NR, 512), w1i)
    m1, dinv = pl.pallas_call(
        _tc_pre,
        out_shape=[_sds((NPR, 128)), _sds((NR, 128))],
    )(t1, degp.reshape(2, NPR, 64), sel)

    s1 = _prop_kernel(m1.reshape(NP, F), srcp, dstp, zc)
    m2 = pl.pallas_call(
        _tc_mid1,
        out_shape=_sds((NPR, 128)),
    )(s1.reshape(2, NPR, 128), m1, dinv, fold128, b1r, g1r, be1r, w2i)

    s2 = _prop_kernel(m2.reshape(NP, F), srcp, dstp, zc)
    m3 = pl.pallas_call(
        _tc_mid2,
        out_shape=_sds((NPR, 128)),
    )(s2.reshape(2, NPR, 128), m2, dinv, fold128, b2r, g2r, be2r)

    s3 = _prop_kernel(m3.reshape(NP, F), srcp, dstp, zc)
    out = pl.pallas_call(
        _tc_fin,
        out_shape=_sds((NR, 512)),
    )(s3.reshape(2, NPR, 128), m3, dinv, fold512, x.reshape(NR, 512), w3i,
      b3r, g3r, be3r)
    return out.reshape(N, 4 * F)


# speedup vs baseline: 1.0164x; 1.0006x over previous
import functools

import jax
import jax.numpy as jnp
from jax import lax
from jax.experimental import pallas as pl
from jax.experimental.pallas import tpu as pltpu
from jax.experimental.pallas import tpu_sc as plsc

N = 10000
NP = 10240
F = 32
NC, NS = 2, 16
NW = NC * NS
CH = 125
NCHUNK = 80
EPT = CH * NCHUNK
NBUF = 8
LEAD = 4
EPS = 1e-5

_mesh = plsc.VectorSubcoreMesh(core_axis_name="c", subcore_axis_name="s")
_sc_params = pltpu.CompilerParams(use_tc_tiling_on_sc=False)


@functools.partial(
    pl.kernel,
    out_type=jax.ShapeDtypeStruct((NC, NP, 16), jnp.float32),
    mesh=_mesh,
    scratch_types=[
        pltpu.VMEM((NCHUNK, CH), jnp.int32),
        pltpu.VMEM((CH, 16), jnp.float32),
        pltpu.VMEM_SHARED((NP, 16), jnp.float32),
    ],
    compiler_params=_sc_params,
)
def _deg_kernel(dst_hbm, z16_hbm, out_hbm, dst_v, ones_v, acc_sh):
    c = lax.axis_index("c")
    s = lax.axis_index("s")
    wid = c * NS + s
    pltpu.sync_copy(dst_hbm.at[wid], dst_v)

    one = jnp.full((16,), 1.0, jnp.float32)

    def ones_body(i, carry):
        ones_v[i] = one
        return carry

    lax.fori_loop(0, CH, ones_body, 0)
    rpt = NP // NS
    pltpu.sync_copy(z16_hbm.at[pl.ds(s * rpt, rpt)], acc_sh.at[pl.ds(s * rpt, rpt)])
    plsc.subcore_barrier()

    def body(j, carry):
        pltpu.sync_copy(ones_v, acc_sh.at[dst_v.at[j]], add=True)
        return carry

    lax.fori_loop(0, NCHUNK, body, 0)
    plsc.subcore_barrier()

    @pl.when(s == 0)
    def _():
        pltpu.sync_copy(acc_sh, out_hbm.at[c])


@functools.partial(
    pl.kernel,
    out_type=jax.ShapeDtypeStruct((NC, NP, F), jnp.float32),
    mesh=_mesh,
    scratch_types=[
        pltpu.VMEM((NCHUNK, CH), jnp.int32),
        pltpu.VMEM((NCHUNK, CH), jnp.int32),
        pltpu.VMEM((NBUF, CH, F), jnp.float32),
        pltpu.VMEM_SHARED((NP, F), jnp.float32),
        pltpu.VMEM_SHARED((NP, F), jnp.float32),
        pltpu.SemaphoreType.DMA((NBUF,)),
        pltpu.SemaphoreType.DMA((NBUF,)),
    ],
    compiler_params=_sc_params,
)
def _prop_kernel(m_hbm, src_hbm, dst_hbm, zc_hbm, out_hbm,
                 src_v, dst_v, rows_v, acc_sh, tab_sh, gsem, ssem):
    c = lax.axis_index("c")
    s = lax.axis_index("s")
    wid = c * NS + s
    pltpu.sync_copy(src_hbm.at[wid], src_v)
    pltpu.sync_copy(dst_hbm.at[wid], dst_v)
    rpt = NP // NS
    pltpu.sync_copy(zc_hbm.at[pl.ds(s * rpt, rpt)], acc_sh.at[pl.ds(s * rpt, rpt)])
    pltpu.sync_copy(m_hbm.at[pl.ds(s * rpt, rpt)], tab_sh.at[pl.ds(s * rpt, rpt)])
    plsc.subcore_barrier()

    def issue(j, b):
        pltpu.async_copy(tab_sh.at[src_v.at[j]], rows_v.at[b], gsem.at[b])

    for b in range(LEAD):
        issue(b, b)

    def body(jj, carry):
        base = jj * NBUF
        for k in range(NBUF):
            j = base + k
            pltpu.make_async_copy(tab_sh.at[src_v.at[0]], rows_v.at[k],
                                  gsem.at[k]).wait()
            pltpu.async_copy(rows_v.at[k], acc_sh.at[dst_v.at[j]], ssem.at[k],
                             add=True)
            nxt = j + LEAD
            bn = (k + LEAD) % NBUF

            @pl.when(nxt < NCHUNK)
            def _():
                @pl.when(j >= NBUF - LEAD)
                def _():
                    pltpu.make_async_copy(rows_v.at[bn],
                                          acc_sh.at[dst_v.at[0]],
                                          ssem.at[bn]).wait()
                issue(nxt, bn)
        return carry

    lax.fori_loop(0, NCHUNK // NBUF, body, 0)
    for b in range(NBUF):
        pltpu.make_async_copy(rows_v.at[b], acc_sh.at[dst_v.at[0]],
                              ssem.at[b]).wait()
    plsc.subcore_barrier()

    @pl.when(s == 0)
    def _():
        pltpu.sync_copy(acc_sh, out_hbm.at[c])


NR = N // 4
NPR = NP // 4


def _tc_mm1(x_ref, w1_ref, t1_ref):
    t1_ref[...] = jnp.dot(x_ref[...], w1_ref[...],
                          preferred_element_type=jnp.float32)


def _tc_pre(t1_ref, dp_ref, sel_ref, m1_ref, dinv_ref):
    d64 = dp_ref[0, 0:NR, :] + dp_ref[1, 0:NR, :]
    deg4 = jnp.dot(d64, sel_ref[...], preferred_element_type=jnp.float32) + 1.0
    dinv = lax.rsqrt(deg4)
    dinv_ref[...] = dinv
    m1_ref[0:NR, :] = dinv * t1_ref[...]


def _bn_relu(p, fold_ref, ga, be):
    mu = jnp.dot(jnp.mean(p, axis=0, keepdims=True), fold_ref[...],
                 preferred_element_type=jnp.float32)
    var = jnp.dot(jnp.mean((p - mu) * (p - mu), axis=0, keepdims=True),
                  fold_ref[...], preferred_element_type=jnp.float32)
    return jnp.maximum((p - mu) * lax.rsqrt(var + EPS) * ga + be, 0.0)


def _tc_mid1(sp_ref, m1_ref, dinv_ref, fold_ref, b_ref, ga_ref, be_ref,
             w2_ref, m2_ref):
    dinv = dinv_ref[...]
    p = dinv * (sp_ref[0, 0:NR, :] + sp_ref[1, 0:NR, :] + m1_ref[0:NR, :]) \
        + b_ref[...]
    h = _bn_relu(p, fold_ref, ga_ref[...], be_ref[...])
    t2 = jnp.dot(h, w2_ref[...], preferred_element_type=jnp.float32)
    m2_ref[0:NR, :] = dinv * t2


def _tc_mid2(sp_ref, m2_ref, dinv_ref, fold_ref, b_ref, ga_ref, be_ref,
             m3_ref):
    dinv = dinv_ref[...]
    p = dinv * (sp_ref[0, 0:NR, :] + sp_ref[1, 0:NR, :] + m2_ref[0:NR, :]) \
        + b_ref[...]
    m3_ref[0:NR, :] = dinv * _bn_relu(p, fold_ref, ga_ref[...], be_ref[...])


def _tc_fin(sp_ref, m3_ref, dinv_ref, fold_ref, x_ref, w3_ref, b_ref, ga_ref,
            be_ref, o_ref):
    dinv = dinv_ref[...]
    q = dinv * (sp_ref[0, 0:NR, :] + sp_ref[1, 0:NR, :] + m3_ref[0:NR, :])
    t3 = jnp.dot(q, w3_ref[...], preferred_element_type=jnp.float32) + b_ref[...]
    mu = jnp.dot(jnp.mean(t3, axis=0, keepdims=True), fold_ref[...],
                 preferred_element_type=jnp.float32)
    var = jnp.dot(jnp.mean((t3 - mu) * (t3 - mu), axis=0, keepdims=True),
                  fold_ref[...], preferred_element_type=jnp.float32)
    bn = (t3 - mu) * lax.rsqrt(var + EPS) * ga_ref[...] + be_ref[...]
    o_ref[...] = jnp.maximum(bn + x_ref[...], 0.0)


def _sds(shape):
    return jax.ShapeDtypeStruct(shape, jnp.float32)


def kernel(x, ei, batch, W1, b1, g1, be1, W2, b2, g2, be2, W3, b3, g3, be3):
    del batch
    eir = ei.astype(jnp.int32).reshape(2, NW, NCHUNK, CH)
    srcp, dstp = eir[0], eir[1]
    zc = jnp.zeros((NP, F), jnp.float32)
    z16 = jnp.zeros((NP, 16), jnp.float32)

    eye4 = jnp.eye(4, dtype=jnp.float32)
    w1i = jnp.kron(eye4, W1)
    w2i = jnp.kron(eye4, W2)
    w3i = jnp.kron(eye4, W3)
    sel = jnp.kron(eye4, jnp.zeros((16, F), jnp.float32).at[0, :].set(1.0))
    quarter = jnp.full((4, 4), 0.25, jnp.float32)
    fold128 = jnp.kron(quarter, jnp.eye(F, dtype=jnp.float32))
    fold512 = jnp.kron(quarter, jnp.eye(4 * F, dtype=jnp.float32))
    b1r, g1r, be1r = (jnp.tile(v, 4).reshape(1, 128) for v in (b1, g1, be1))
    b2r, g2r, be2r = (jnp.tile(v, 4).reshape(1, 128) for v in (b2, g2, be2))
    b3r, g3r, be3r = (jnp.tile(v, 4).reshape(1, 512) for v in (b3, g3, be3))

    degp = _deg_kernel(dstp, z16)
    t1 = pl.pallas_call(_tc_mm1, out_shape=_sds((NR, 128)))(
        x.reshape(NR, 512), w1i)
    m1, dinv = pl.pallas_call(
        _tc_pre,
        out_shape=[_sds((NPR, 128)), _sds((NR, 128))],
    )(t1, degp.reshape(2, NPR, 64), sel)

    s1 = _prop_kernel(m1.reshape(NP, F), srcp, dstp, zc)
    m2 = pl.pallas_call(
        _tc_mid1,
        out_shape=_sds((NPR, 128)),
    )(s1.reshape(2, NPR, 128), m1, dinv, fold128, b1r, g1r, be1r, w2i)

    s2 = _prop_kernel(m2.reshape(NP, F), srcp, dstp, zc)
    m3 = pl.pallas_call(
        _tc_mid2,
        out_shape=_sds((NPR, 128)),
    )(s2.reshape(2, NPR, 128), m2, dinv, fold128, b2r, g2r, be2r)

    s3 = _prop_kernel(m3.reshape(NP, F), srcp, dstp, zc)
    out = pl.pallas_call(
        _tc_fin,
        out_shape=_sds((NR, 512)),
    )(s3.reshape(2, NPR, 128), m3, dinv, fold512, x.reshape(NR, 512), w3i,
      b3r, g3r, be3r)
    return out.reshape(N, 4 * F)
